# baseline (speedup 1.0000x reference)
import jax
import jax.numpy as jnp
from jax.experimental import pallas as pl
from jax.experimental.pallas import tpu as pltpu

_WC = 16384
_RB = 8
_ND = 131072
_TPS = _WC // 128


def _argmax_body(prob_ref, g_ref, s_ref, o_ref, tl_ref):
    p = prob_ref[...]
    idx = jnp.argmax(p, axis=1).astype(jnp.int32)
    t = idx * 16
    g_ref[...] = (idx // 8)[:, None]
    s_ref[...] = (idx % 8)[:, None]
    o_ref[...] = (t % 128)[:, None]
    tl_ref[...] = (t // 128)[:, None]


def _patch_body(g_ref, s_ref, o_ref, *refs):
    sig_refs = refs[:_RB]
    p_ref = refs[_RB]
    i = pl.program_id(0)
    sub = jax.lax.broadcasted_iota(jnp.int32, (8, 16), 0)
    k_iota = jax.lax.broadcasted_iota(jnp.int32, (16, 128), 0)
    l_iota = jax.lax.broadcasted_iota(jnp.int32, (16, 128), 1)
    smat = (l_iota % 16 == k_iota).astype(jnp.float32)
    lane = jax.lax.broadcasted_iota(jnp.int32, (1, 128), 1)
    for r in range(_RB):
        s_r = s_ref[i * _RB + r]
        o_r = o_ref[i * _RB + r]
        sel = (sub == s_r).astype(jnp.float32)
        v = jnp.sum(sig_refs[r][0] * sel, axis=0, keepdims=True)
        vt = jax.lax.dot(v, smat, precision=jax.lax.Precision.HIGHEST)
        w = (lane >= o_r) & (lane < o_r + 16)
        p_ref[pl.ds(r, 1), :] = jnp.where(w, vt, 0.0)


def _write_body(p_ref, tl_ref, out_ref):
    i = pl.program_id(0)
    c = pl.program_id(1)
    out_ref[...] = jnp.zeros_like(out_ref)
    for r in range(_RB):
        t_r = tl_ref[i * _RB + r, 0]

        @pl.when(t_r // _TPS == c)
        def _patch(r=r, t_r=t_r):
            col = pl.multiple_of((t_r % _TPS) * 128, 128)
            out_ref[pl.ds(r, 1), pl.ds(col, 128)] = p_ref[pl.ds(r, 1), :]


def kernel(signals, prob):
    B, N, D = signals.shape

    g, s, o, tl = pl.pallas_call(
        _argmax_body,
        grid=(B // 16,),
        in_specs=[pl.BlockSpec((16, N), lambda i: (i, 0))],
        out_specs=tuple(pl.BlockSpec((16, 1), lambda i: (i, 0)) for _ in range(4)),
        out_shape=tuple(
            jax.ShapeDtypeStruct((B, 1), jnp.int32) for _ in range(4)
        ),
    )(prob)

    sig_specs = [
        pl.BlockSpec(
            (1, 8, D),
            (lambda i, g, s, o, r=r: (i * _RB + r, g[i * _RB + r], 0)),
        )
        for r in range(_RB)
    ]
    patches = pl.pallas_call(
        _patch_body,
        grid_spec=pltpu.PrefetchScalarGridSpec(
            num_scalar_prefetch=3,
            grid=(B // _RB,),
            in_specs=sig_specs,
            out_specs=pl.BlockSpec((_RB, 128), lambda i, g, s, o: (i, 0)),
        ),
        out_shape=jax.ShapeDtypeStruct((B, 128), jnp.float32),
    )(g.reshape(B), s.reshape(B), o.reshape(B), *([signals] * _RB))

    out = pl.pallas_call(
        _write_body,
        grid=(B // _RB, _ND // _WC),
        in_specs=[
            pl.BlockSpec((_RB, 128), lambda i, c: (i, 0)),
            pl.BlockSpec(memory_space=pltpu.SMEM),
        ],
        out_specs=pl.BlockSpec((_RB, _WC), lambda i, c: (i, c)),
        out_shape=jax.ShapeDtypeStruct((B, _ND), jnp.float32),
    )(patches, tl)
    return out


# speedup vs baseline: 1.0348x; 1.0348x over previous
import jax
import jax.numpy as jnp
from jax.experimental import pallas as pl
from jax.experimental.pallas import tpu as pltpu

_WC = 16384
_RB = 8
_ND = 131072
_TPS = _WC // 128
_B = 128


def _argmax_body(prob_ref, g_ref, s_ref, o_ref, tl_ref):
    p = prob_ref[...]
    idx = jnp.argmax(p, axis=1).astype(jnp.int32)
    t = idx * 16
    g_ref[...] = (idx // 8)[:, None]
    s_ref[...] = (idx % 8)[:, None]
    o_ref[...] = (t % 128)[:, None]
    tl_ref[...] = (t // 128)[:, None]


def _patch_body(g_ref, s_ref, o_ref, sig_ref, p_ref, grp_ref, sem):
    copies = []
    for b in range(_B):
        gb = g_ref[b, 0]
        cp = pltpu.make_async_copy(
            sig_ref.at[b, pl.ds(gb * 8, 8), :], grp_ref.at[b], sem
        )
        cp.start()
        copies.append(cp)
    for cp in copies:
        cp.wait()

    data = grp_ref[...]
    sub = jax.lax.broadcasted_iota(jnp.int32, (_B, 8, 16), 1)
    sel = (sub == s_ref[...][:, :, None]).astype(jnp.float32)
    v = jnp.sum(data * sel, axis=1)

    k_iota = jax.lax.broadcasted_iota(jnp.int32, (16, 128), 0)
    l_iota = jax.lax.broadcasted_iota(jnp.int32, (16, 128), 1)
    smat = (l_iota % 16 == k_iota).astype(jnp.float32)
    vt = jax.lax.dot(v, smat, precision=jax.lax.Precision.HIGHEST)

    lane = jax.lax.broadcasted_iota(jnp.int32, (_B, 128), 1)
    o = o_ref[...]
    p_ref[...] = jnp.where((lane >= o) & (lane < o + 16), vt, 0.0)


def _write_body(p_ref, tl_ref, out_ref):
    i = pl.program_id(0)
    c = pl.program_id(1)
    out_ref[...] = jnp.zeros_like(out_ref)
    for r in range(_RB):
        t_r = tl_ref[i * _RB + r, 0]

        @pl.when(t_r // _TPS == c)
        def _patch(r=r, t_r=t_r):
            col = pl.multiple_of((t_r % _TPS) * 128, 128)
            out_ref[pl.ds(r, 1), pl.ds(col, 128)] = p_ref[pl.ds(r, 1), :]


def kernel(signals, prob):
    B, N, D = signals.shape

    g, s, o, tl = pl.pallas_call(
        _argmax_body,
        grid=(B // 16,),
        in_specs=[pl.BlockSpec((16, N), lambda i: (i, 0))],
        out_specs=tuple(pl.BlockSpec((16, 1), lambda i: (i, 0)) for _ in range(4)),
        out_shape=tuple(
            jax.ShapeDtypeStruct((B, 1), jnp.int32) for _ in range(4)
        ),
    )(prob)

    patches = pl.pallas_call(
        _patch_body,
        in_specs=[
            pl.BlockSpec(memory_space=pltpu.SMEM),
            pl.BlockSpec((B, 1), lambda: (0, 0)),
            pl.BlockSpec((B, 1), lambda: (0, 0)),
            pl.BlockSpec(memory_space=pl.ANY),
        ],
        out_specs=pl.BlockSpec((B, 128), lambda: (0, 0)),
        out_shape=jax.ShapeDtypeStruct((B, 128), jnp.float32),
        scratch_shapes=[
            pltpu.VMEM((B, 8, D), jnp.float32),
            pltpu.SemaphoreType.DMA,
        ],
    )(g, s, o, signals)

    out = pl.pallas_call(
        _write_body,
        grid=(B // _RB, _ND // _WC),
        in_specs=[
            pl.BlockSpec((_RB, 128), lambda i, c: (i, 0)),
            pl.BlockSpec(memory_space=pltpu.SMEM),
        ],
        out_specs=pl.BlockSpec((_RB, _WC), lambda i, c: (i, c)),
        out_shape=jax.ShapeDtypeStruct((B, _ND), jnp.float32),
    )(patches, tl)
    return out


# speedup vs baseline: 1.1749x; 1.1353x over previous
import jax
import jax.numpy as jnp
from jax.experimental import pallas as pl
from jax.experimental.pallas import tpu as pltpu

_WC = 16384
_RB = 8
_ND = 131072
_TPS = _WC // 128
_B = 128


def _argmax_body(prob_ref, g_ref, s_ref, o_ref, tl_ref):
    p = prob_ref[...]
    idx = jnp.argmax(p, axis=1).astype(jnp.int32)
    t = idx * 16
    g_ref[...] = (idx // 8)[:, None]
    s_ref[...] = (idx % 8)[:, None]
    o_ref[...] = (t % 128)[:, None]
    tl_ref[...] = (t // 128)[:, None]


def _patch_body(g_ref, s_ref, o_ref, sig_ref, p_ref, grp_ref, sem):
    copies = []
    for b in range(_B):
        gb = g_ref[b, 0]
        cp = pltpu.make_async_copy(
            sig_ref.at[b, pl.ds(gb * 8, 8), :], grp_ref.at[b], sem
        )
        cp.start()
        copies.append(cp)
    for cp in copies:
        cp.wait()

    data = grp_ref[...]
    sub = jax.lax.broadcasted_iota(jnp.int32, (_B, 8, 16), 1)
    sel = (sub == s_ref[...][:, :, None]).astype(jnp.float32)
    v = jnp.sum(data * sel, axis=1)

    k_iota = jax.lax.broadcasted_iota(jnp.int32, (16, 128), 0)
    l_iota = jax.lax.broadcasted_iota(jnp.int32, (16, 128), 1)
    smat = (l_iota % 16 == k_iota).astype(jnp.float32)
    vt = jax.lax.dot(v, smat, precision=jax.lax.Precision.HIGHEST)

    lane = jax.lax.broadcasted_iota(jnp.int32, (_B, 128), 1)
    o = o_ref[...]
    p_ref[...] = jnp.where((lane >= o) & (lane < o + 16), vt, 0.0)


def _write_body(p_ref, tl_ref, out_ref):
    i = pl.program_id(0)
    out_ref[...] = jnp.zeros_like(out_ref)
    for r in range(_RB):
        t_r = tl_ref[i * _RB + r, 0]
        col = pl.multiple_of(t_r * 128, 128)
        out_ref[pl.ds(r, 1), pl.ds(col, 128)] = p_ref[pl.ds(r, 1), :]


def kernel(signals, prob):
    B, N, D = signals.shape

    g, s, o, tl = pl.pallas_call(
        _argmax_body,
        grid=(B // 16,),
        in_specs=[pl.BlockSpec((16, N), lambda i: (i, 0))],
        out_specs=tuple(pl.BlockSpec((16, 1), lambda i: (i, 0)) for _ in range(4)),
        out_shape=tuple(
            jax.ShapeDtypeStruct((B, 1), jnp.int32) for _ in range(4)
        ),
    )(prob)

    patches = pl.pallas_call(
        _patch_body,
        in_specs=[
            pl.BlockSpec(memory_space=pltpu.SMEM),
            pl.BlockSpec((B, 1), lambda: (0, 0)),
            pl.BlockSpec((B, 1), lambda: (0, 0)),
            pl.BlockSpec(memory_space=pl.ANY),
        ],
        out_specs=pl.BlockSpec((B, 128), lambda: (0, 0)),
        out_shape=jax.ShapeDtypeStruct((B, 128), jnp.float32),
        scratch_shapes=[
            pltpu.VMEM((B, 8, D), jnp.float32),
            pltpu.SemaphoreType.DMA,
        ],
    )(g, s, o, signals)

    out = pl.pallas_call(
        _write_body,
        grid=(B // _RB,),
        in_specs=[
            pl.BlockSpec((_RB, 128), lambda i: (i, 0)),
            pl.BlockSpec(memory_space=pltpu.SMEM),
        ],
        out_specs=pl.BlockSpec((_RB, _ND), lambda i: (i, 0)),
        out_shape=jax.ShapeDtypeStruct((B, _ND), jnp.float32),
    )(patches, tl)
    return out


# speedup vs baseline: 10.5772x; 9.0029x over previous
import jax
import jax.numpy as jnp
from jax.experimental import pallas as pl
from jax.experimental.pallas import tpu as pltpu

_RB = 8
_ND = 131072
_B = 128


def _argmax_body(prob_ref, qt_ref, ql_ref, o_ref, tl_ref):
    p = prob_ref[...]
    idx = jnp.argmax(p, axis=1).astype(jnp.int32)
    qt_ref[...] = (idx // 128)[:, None]
    ql_ref[...] = (idx % 128)[:, None]
    o_ref[...] = ((idx % 8) * 16)[:, None]
    tl_ref[...] = (idx // 8)[:, None]


def _patch_body(qt_ref, ql_ref, o_ref, sig_ref, p_ref, blk_ref, sem):
    copies = []
    for b in range(_B):
        qtb = qt_ref[b, 0]
        cp = pltpu.make_async_copy(
            sig_ref.at[b, :, pl.ds(qtb * 128, 128)], blk_ref.at[b], sem
        )
        cp.start()
        copies.append(cp)
    for cp in copies:
        cp.wait()

    blk = blk_ref[...]
    lane = jax.lax.broadcasted_iota(jnp.int32, (_B, 128), 1)
    oh = (lane == ql_ref[...]).astype(jnp.float32)[:, None, :]
    v = jnp.sum(blk * oh, axis=2)

    k_iota = jax.lax.broadcasted_iota(jnp.int32, (16, 128), 0)
    l_iota = jax.lax.broadcasted_iota(jnp.int32, (16, 128), 1)
    smat = (l_iota % 16 == k_iota).astype(jnp.float32)
    vt = jax.lax.dot(v, smat, precision=jax.lax.Precision.HIGHEST)

    o = o_ref[...]
    p_ref[...] = jnp.where((lane >= o) & (lane < o + 16), vt, 0.0)


def _write_body(p_ref, tl_ref, out_ref):
    i = pl.program_id(0)
    out_ref[...] = jnp.zeros_like(out_ref)
    for r in range(_RB):
        t_r = tl_ref[i * _RB + r, 0]
        col = pl.multiple_of(t_r * 128, 128)
        out_ref[pl.ds(r, 1), pl.ds(col, 128)] = p_ref[pl.ds(r, 1), :]


def kernel(signals, prob):
    B, N, D = signals.shape
    sig_t = jnp.transpose(signals, (0, 2, 1))

    qt, ql, o, tl = pl.pallas_call(
        _argmax_body,
        grid=(B // 16,),
        in_specs=[pl.BlockSpec((16, N), lambda i: (i, 0))],
        out_specs=tuple(pl.BlockSpec((16, 1), lambda i: (i, 0)) for _ in range(4)),
        out_shape=tuple(
            jax.ShapeDtypeStruct((B, 1), jnp.int32) for _ in range(4)
        ),
    )(prob)

    patches = pl.pallas_call(
        _patch_body,
        in_specs=[
            pl.BlockSpec(memory_space=pltpu.SMEM),
            pl.BlockSpec((B, 1), lambda: (0, 0)),
            pl.BlockSpec((B, 1), lambda: (0, 0)),
            pl.BlockSpec(memory_space=pl.ANY),
        ],
        out_specs=pl.BlockSpec((B, 128), lambda: (0, 0)),
        out_shape=jax.ShapeDtypeStruct((B, 128), jnp.float32),
        scratch_shapes=[
            pltpu.VMEM((B, D, 128), jnp.float32),
            pltpu.SemaphoreType.DMA,
        ],
    )(qt, ql, o, sig_t)

    out = pl.pallas_call(
        _write_body,
        grid=(B // _RB,),
        in_specs=[
            pl.BlockSpec((_RB, 128), lambda i: (i, 0)),
            pl.BlockSpec(memory_space=pltpu.SMEM),
        ],
        out_specs=pl.BlockSpec((_RB, _ND), lambda i: (i, 0)),
        out_shape=jax.ShapeDtypeStruct((B, _ND), jnp.float32),
    )(patches, tl)
    return out
